# baseline (speedup 1.0000x reference)
import functools

import jax
import jax.numpy as jnp
from jax import lax
from jax.experimental import pallas as pl
from jax.experimental.pallas import tpu as pltpu
from jax.experimental.pallas import tpu_sc as plsc

N_VOCAB = 1000000
D_EMB = 64
BATCH = 4
SEQ = 4096
B_TOTAL = BATCH * SEQ

_info = plsc.get_sparse_core_info()
_NC = _info.num_cores
_NS = _info.num_subcores
_NW = _NC * _NS
_BPW = B_TOTAL // _NW
_LANES = 16
_VPR = D_EMB // _LANES


def _make_sc_kernel():
    mesh = plsc.VectorSubcoreMesh(core_axis_name="c", subcore_axis_name="s")

    @functools.partial(
        pl.kernel,
        mesh=mesh,
        compiler_params=pltpu.CompilerParams(use_tc_tiling_on_sc=False),
        out_type=jax.ShapeDtypeStruct((B_TOTAL, D_EMB), jnp.float32),
        scratch_types=[
            pltpu.VMEM((_BPW,), jnp.int32),
            pltpu.VMEM((_BPW, D_EMB), jnp.float32),
            pltpu.VMEM((_BPW, D_EMB), jnp.float32),
            pltpu.SemaphoreType.DMA,
        ],
    )
    def emb_kernel(ids_hbm, table_hbm, pe_hbm, out_hbm, idx_v, rows_v, pe_v, sem):
        wid = lax.axis_index("s") * _NC + lax.axis_index("c")
        base = wid * _BPW
        pos_base = lax.rem(base, SEQ)

        pltpu.sync_copy(ids_hbm.at[pl.ds(base, _BPW)], idx_v)
        gather = pltpu.async_copy(table_hbm.at[idx_v], rows_v, sem)
        pltpu.sync_copy(pe_hbm.at[pl.ds(pos_base, _BPW)], pe_v)
        gather.wait()

        def body(r, carry):
            for j in range(_VPR):
                sl = pl.ds(j * _LANES, _LANES)
                rows_v[r, sl] = (rows_v[r, sl] + pe_v[r, sl]) * 8.0
            return carry

        lax.fori_loop(0, _BPW, body, 0, unroll=4)

        pltpu.sync_copy(rows_v, out_hbm.at[pl.ds(base, _BPW)])

    return emb_kernel


_emb_kernel = _make_sc_kernel()


@jax.jit
def kernel(input_ids, w, pos_encoding):
    flat_ids = input_ids.reshape(-1)
    pe2d = pos_encoding.reshape(pos_encoding.shape[1], D_EMB)
    out = _emb_kernel(flat_ids, w, pe2d)
    return out.reshape(BATCH, SEQ, D_EMB)


# speedup vs baseline: 1.7138x; 1.7138x over previous
import functools

import jax
import jax.numpy as jnp
from jax import lax
from jax.experimental import pallas as pl
from jax.experimental.pallas import tpu as pltpu
from jax.experimental.pallas import tpu_sc as plsc

N_VOCAB = 1000000
D_EMB = 64
BATCH = 4
SEQ = 4096
B_TOTAL = BATCH * SEQ

_info = plsc.get_sparse_core_info()
_NC = _info.num_cores
_NS = _info.num_subcores
_NW = _NC * _NS
_BPW = B_TOTAL // _NW
_LANES = 16
_VPR = D_EMB // _LANES
_HALF = _BPW // 2


def _make_sc_kernel():
    mesh = plsc.VectorSubcoreMesh(core_axis_name="c", subcore_axis_name="s")

    @functools.partial(
        pl.kernel,
        mesh=mesh,
        compiler_params=pltpu.CompilerParams(needs_layout_passes=False),
        out_type=jax.ShapeDtypeStruct((B_TOTAL, D_EMB), jnp.float32),
        scratch_types=[
            pltpu.VMEM((_BPW,), jnp.int32),
            pltpu.VMEM((_BPW, D_EMB), jnp.float32),
            pltpu.VMEM((_HALF, D_EMB), jnp.float32),
            pltpu.SemaphoreType.DMA,
            pltpu.SemaphoreType.DMA,
        ],
    )
    def emb_kernel(ids_hbm, table_hbm, pe_hbm, out_hbm,
                   ids_v, rows_v, pe_v, sem, pe_sem):
        wid = lax.axis_index("s") * _NC + lax.axis_index("c")
        base = wid * _BPW
        pos_base = lax.rem(base, SEQ)

        pltpu.sync_copy(ids_hbm.at[pl.ds(base, _BPW)], ids_v)

        iota = lax.iota(jnp.int32, _LANES)

        def fire(gi, carry):
            v = ids_v[pl.ds(gi * _LANES, _LANES)]
            for l in range(_LANES):
                rid = jnp.sum(jnp.where(iota == l, v, 0))
                pltpu.async_copy(
                    table_hbm.at[pl.ds(rid, 1)],
                    rows_v.at[pl.ds(gi * _LANES + l, 1)], sem)
            return carry

        lax.fori_loop(0, _BPW // _LANES, fire, 0)

        pe_dma = pltpu.async_copy(
            pe_hbm.at[pl.ds(pos_base, _HALF)], pe_v, pe_sem)

        pltpu.make_async_copy(
            table_hbm.at[pl.ds(0, _BPW)], rows_v, sem).wait()
        pe_dma.wait()

        def compute0(r, carry):
            for j in range(_VPR):
                sl = pl.ds(j * _LANES, _LANES)
                rows_v[r, sl] = (rows_v[r, sl] + pe_v[r, sl]) * 8.0
            return carry

        lax.fori_loop(0, _HALF, compute0, 0)

        pltpu.sync_copy(pe_hbm.at[pl.ds(pos_base + _HALF, _HALF)], pe_v)

        def compute1(r, carry):
            for j in range(_VPR):
                sl = pl.ds(j * _LANES, _LANES)
                rows_v[_HALF + r, sl] = (rows_v[_HALF + r, sl]
                                         + pe_v[r, sl]) * 8.0
            return carry

        lax.fori_loop(0, _HALF, compute1, 0)

        pltpu.sync_copy(rows_v, out_hbm.at[pl.ds(base, _BPW)])

    return emb_kernel


_emb_kernel = _make_sc_kernel()


@jax.jit
def kernel(input_ids, w, pos_encoding):
    flat_ids = input_ids.reshape(-1)
    pe2d = pos_encoding.reshape(pos_encoding.shape[1], D_EMB)
    out = _emb_kernel(flat_ids, w, pe2d)
    return out.reshape(BATCH, SEQ, D_EMB)
